# baseline (speedup 1.0000x reference)
import functools

import jax
import jax.numpy as jnp
from jax import lax
from jax.experimental import pallas as pl
from jax.experimental.pallas import tpu as pltpu
from jax.experimental.pallas import tpu_sc as plsc

_N = 18432
_K = 1024
_D = 64
_BLOCK = 3072
_COMMIT = 0.25

_NSEG = 2
_SEG = _N // _NSEG

_NC = 2
_NS = 16
_NW = _NC * _NS
_BPW = _SEG // _NW
_CHUNK = 96
_NCHUNK = _BPW // _CHUNK


def _vq_block(z_ref, embm2_ref, z2_ref, e2_ref, iotaf_ref,
              idx_ref, part_ref):
    z = z_ref[...]
    mm2 = jax.lax.dot_general(z, embm2_ref[...], (((1,), (1,)), ((), ())),
                              preferred_element_type=jnp.float32)
    d = (z2_ref[...] + e2_ref[...]) + mm2
    m = jnp.min(d, axis=1, keepdims=True)
    iotaf = iotaf_ref[...]
    cand = jnp.where(d == m, iotaf, jnp.float32(_K))
    idxf = jnp.min(cand, axis=1, keepdims=True)
    idx_ref[...] = idxf[:, 0].astype(jnp.int32)
    part_ref[...] = jnp.sum(m)[None, None, None]


def _tc_segment(z_seg, embm2, z2_seg, e2, iotaf):
    nblocks = _SEG // _BLOCK
    return pl.pallas_call(
        _vq_block,
        grid=(nblocks,),
        in_specs=[
            pl.BlockSpec((_BLOCK, _D), lambda i: (i, 0)),
            pl.BlockSpec((_K, _D), lambda i: (0, 0)),
            pl.BlockSpec((_BLOCK, 1), lambda i: (i, 0)),
            pl.BlockSpec((1, _K), lambda i: (0, 0)),
            pl.BlockSpec((1, _K), lambda i: (0, 0)),
        ],
        out_specs=[
            pl.BlockSpec((_BLOCK,), lambda i: (i,)),
            pl.BlockSpec((1, 1, 1), lambda i: (i, 0, 0)),
        ],
        out_shape=[
            jax.ShapeDtypeStruct((_SEG,), jnp.int32),
            jax.ShapeDtypeStruct((nblocks, 1, 1), jnp.float32),
        ],
        compiler_params=pltpu.CompilerParams(
            dimension_semantics=("parallel",)),
    )(z_seg, embm2, z2_seg, e2, iotaf)


@functools.partial(
    pl.kernel,
    mesh=plsc.VectorSubcoreMesh(core_axis_name="c", subcore_axis_name="s"),
    compiler_params=pltpu.CompilerParams(use_tc_tiling_on_sc=False),
    out_type=jax.ShapeDtypeStruct((_SEG, _D), jnp.float32),
    scratch_types=[
        pltpu.VMEM((_NCHUNK, _CHUNK), jnp.int32),
        pltpu.VMEM((_BPW, _D), jnp.float32),
        pltpu.SemaphoreType.DMA,
    ],
)
def _sc_gather(emb_hbm, idx_hbm, out_hbm, idx_v, rows_v, sem):
    wid = lax.axis_index("s") * _NC + lax.axis_index("c")
    base = wid * _BPW
    for j in range(_NCHUNK):
        pltpu.sync_copy(idx_hbm.at[pl.ds(base + j * _CHUNK, _CHUNK)],
                        idx_v.at[j])
    copies = [
        pltpu.async_copy(emb_hbm.at[idx_v.at[j]],
                         rows_v.at[pl.ds(j * _CHUNK, _CHUNK)], sem)
        for j in range(_NCHUNK)
    ]
    for c in copies:
        c.wait()
    pltpu.sync_copy(rows_v, out_hbm.at[pl.ds(base, _BPW)])


def kernel(z_flat, emb):
    z2 = jnp.sum(z_flat ** 2, axis=1, keepdims=True)
    e2 = jnp.sum(emb ** 2, axis=1)[None, :]
    embm2 = -2.0 * emb
    iotaf = jnp.arange(_K, dtype=jnp.float32)[None, :]
    idxs, qs, parts = [], [], []
    for s in range(_NSEG):
        sl = slice(s * _SEG, (s + 1) * _SEG)
        idx_s, part_s = _tc_segment(z_flat[sl], embm2, z2[sl], e2, iotaf)
        qs.append(_sc_gather(emb, idx_s))
        idxs.append(idx_s)
        parts.append(part_s)
    idx = jnp.concatenate(idxs)
    q = jnp.concatenate(qs, axis=0)
    loss = (jnp.sum(jnp.stack([jnp.sum(p) for p in parts]))
            * ((1.0 + _COMMIT) / (_N * _D)))
    return (loss, q, idx)


# speedup vs baseline: 1.3337x; 1.3337x over previous
import jax
import jax.numpy as jnp
from jax.experimental import pallas as pl
from jax.experimental.pallas import tpu as pltpu

_N = 18432
_K = 1024
_D = 64
_BLOCK = 3072
_COMMIT = 0.25


def _vq_block(z_ref, emb_ref, embm2_ref, z2_ref, e2_ref, iotaf_ref,
              idx_ref, q_ref, part_ref):
    z = z_ref[...]
    mm2 = jax.lax.dot_general(z, embm2_ref[...], (((1,), (1,)), ((), ())),
                              preferred_element_type=jnp.float32)
    d = (z2_ref[...] + e2_ref[...]) + mm2
    m = jnp.min(d, axis=1, keepdims=True)
    iotaf = iotaf_ref[...]
    cand = jnp.where(d == m, iotaf, jnp.float32(_K))
    idxf = jnp.min(cand, axis=1, keepdims=True)
    idx_ref[...] = idxf[:, 0].astype(jnp.int32)
    onehot = (iotaf == idxf).astype(jnp.float32)
    q = jax.lax.dot_general(onehot, emb_ref[...], (((1,), (0,)), ((), ())),
                            preferred_element_type=jnp.float32)
    q_ref[...] = q
    part_ref[...] = jnp.sum(m)[None, None, None]


def kernel(z_flat, emb):
    z2 = jnp.sum(z_flat ** 2, axis=1, keepdims=True)
    e2 = jnp.sum(emb ** 2, axis=1)[None, :]
    nblocks = _N // _BLOCK
    idx, q, part = pl.pallas_call(
        _vq_block,
        grid=(nblocks,),
        in_specs=[
            pl.BlockSpec((_BLOCK, _D), lambda i: (i, 0)),
            pl.BlockSpec((_K, _D), lambda i: (0, 0)),
            pl.BlockSpec((_K, _D), lambda i: (0, 0)),
            pl.BlockSpec((_BLOCK, 1), lambda i: (i, 0)),
            pl.BlockSpec((1, _K), lambda i: (0, 0)),
            pl.BlockSpec((1, _K), lambda i: (0, 0)),
        ],
        out_specs=[
            pl.BlockSpec((_BLOCK,), lambda i: (i,)),
            pl.BlockSpec((_BLOCK, _D), lambda i: (i, 0)),
            pl.BlockSpec((1, 1, 1), lambda i: (i, 0, 0)),
        ],
        out_shape=[
            jax.ShapeDtypeStruct((_N,), jnp.int32),
            jax.ShapeDtypeStruct((_N, _D), jnp.float32),
            jax.ShapeDtypeStruct((nblocks, 1, 1), jnp.float32),
        ],
        compiler_params=pltpu.CompilerParams(
            dimension_semantics=("parallel",)),
    )(z_flat, emb, -2.0 * emb, z2, e2,
      jnp.arange(_K, dtype=jnp.float32)[None, :])
    loss = jnp.sum(part) * ((1.0 + _COMMIT) / (_N * _D))
    return (loss, q, idx)
